# baseline (speedup 1.0000x reference)
import jax
import jax.numpy as jnp
from jax.experimental import pallas as pl


def _gcn(x, edge_index, W, b):
    n = x.shape[0]
    loop = jnp.arange(n, dtype=edge_index.dtype)
    src = jnp.concatenate([edge_index[0], loop])
    dst = jnp.concatenate([edge_index[1], loop])
    xw = x @ W
    deg = jnp.zeros((n,), dtype=x.dtype).at[dst].add(1.0)
    dinv = jax.lax.rsqrt(deg)
    norm = dinv[src] * dinv[dst]
    msg = xw[src] * norm[:, None]
    out = jnp.zeros((n, W.shape[1]), dtype=x.dtype).at[dst].add(msg)
    return out + b


def _head_kernel(p_ref, w_ref, b_ref, o_ref):
    o_ref[...] = jax.nn.relu(p_ref[...] @ w_ref[...] + b_ref[...])


def kernel(x, edge_index, batch, W0, b0, W1, b1, W2, b2, Wfc, bfc):
    h = jax.nn.relu(_gcn(x, edge_index, W0, b0))
    h = jax.nn.relu(_gcn(h, edge_index, W1, b1))
    h = jax.nn.relu(_gcn(h, edge_index, W2, b2))
    summed = jax.ops.segment_sum(h, batch, num_segments=16)
    counts = jax.ops.segment_sum(jnp.ones((h.shape[0], 1), h.dtype), batch, num_segments=16)
    pooled = summed / jnp.maximum(counts, 1.0)
    return pl.pallas_call(
        _head_kernel,
        out_shape=jax.ShapeDtypeStruct((16, Wfc.shape[1]), x.dtype),
    )(pooled, Wfc, bfc.reshape(1, -1))


# speedup vs baseline: 9.0272x; 9.0272x over previous
import jax
import jax.numpy as jnp
from jax import lax
from jax.experimental import pallas as pl
from jax.experimental.pallas import tpu as pltpu
from jax.experimental.pallas import tpu_sc as plsc

N = 50000
B = 16
HID = 64
EMB = 128
NQ = 4
QW = 16
RBLK = 512
N_PAD = 50176
GRID = N_PAD // RBLK
E_PAD = 802816
CHUNK = 128
BLK = 4
ROWS_PER_TILE = N_PAD // 16
CHUNKS_PER_TILE = (E_PAD // CHUNK) // 16
EITERS = CHUNKS_PER_TILE // BLK
ZROWS = 1024


def _edge_acc_body(table, src_idx, dst_idx, acc_out,
                   srcb, dstb, rows, zbuf, acc_sh, gsem):
    c = lax.axis_index("c")
    s = lax.axis_index("s")
    zv = jnp.zeros((QW,), jnp.float32)

    @pl.loop(0, ZROWS)
    def _(i):
        zbuf[i, pl.ds(0, QW)] = zv

    row0 = s * ROWS_PER_TILE
    rem = ROWS_PER_TILE - 3 * ZROWS
    chunk0 = s * CHUNKS_PER_TILE

    for phase in range(2):
        q = c * 2 + phase
        for k in range(3):
            pltpu.sync_copy(zbuf, acc_sh.at[pl.ds(row0 + k * ZROWS, ZROWS)])
        pltpu.sync_copy(zbuf.at[pl.ds(0, rem)],
                        acc_sh.at[pl.ds(row0 + 3 * ZROWS, rem)])
        plsc.subcore_barrier()

        table_q = table.at[q]

        @pl.loop(0, EITERS)
        def _(it):
            cb = chunk0 + it * BLK
            pltpu.sync_copy(src_idx.at[pl.ds(cb, BLK)], srcb)
            pltpu.sync_copy(dst_idx.at[pl.ds(cb, BLK)], dstb)
            descs = [pltpu.async_copy(table_q.at[srcb.at[j]],
                                      rows.at[pl.ds(j * CHUNK, CHUNK)], gsem)
                     for j in range(BLK)]
            for d in descs:
                d.wait()
            for j in range(BLK):
                pltpu.sync_copy(rows.at[pl.ds(j * CHUNK, CHUNK)],
                                acc_sh.at[dstb.at[j]], add=True)

        plsc.subcore_barrier()
        for k in range(3):
            pltpu.sync_copy(acc_sh.at[pl.ds(row0 + k * ZROWS, ZROWS)],
                            acc_out.at[q, pl.ds(row0 + k * ZROWS, ZROWS)])
        pltpu.sync_copy(acc_sh.at[pl.ds(row0 + 3 * ZROWS, rem)],
                        acc_out.at[q, pl.ds(row0 + 3 * ZROWS, rem)])
        plsc.subcore_barrier()


def _edge_accumulate(table, src2d, dst2d):
    f = pl.kernel(
        _edge_acc_body,
        out_type=jax.ShapeDtypeStruct((NQ, N_PAD, QW), jnp.float32),
        mesh=plsc.VectorSubcoreMesh(core_axis_name="c", subcore_axis_name="s"),
        scratch_types=[
            pltpu.VMEM((BLK, CHUNK), jnp.int32),
            pltpu.VMEM((BLK, CHUNK), jnp.int32),
            pltpu.VMEM((BLK * CHUNK, QW), jnp.float32),
            pltpu.VMEM((ZROWS, QW), jnp.float32),
            pltpu.VMEM_SHARED((N_PAD, QW), jnp.float32),
            pltpu.SemaphoreType.DMA,
        ],
        compiler_params=pltpu.CompilerParams(use_tc_tiling_on_sc=False),
    )
    return f(table, src2d, dst2d)


def _to_quarters(y):
    return y.reshape(N_PAD, NQ, QW).transpose(1, 0, 2)


def _from_quarters(a4):
    return a4.transpose(1, 0, 2).reshape(N_PAD, HID)


def _y0_body(x_ref, deg_ref, w_ref, y_ref, dinv_ref):
    xw = jnp.dot(x_ref[...], w_ref[...])
    dinv = lax.rsqrt(deg_ref[...] + 1.0)
    y_ref[...] = xw * dinv
    dinv_ref[...] = dinv


def _mid_body(acc_ref, y_ref, dinv_ref, b_ref, w_ref, o_ref):
    dinv = dinv_ref[...]
    h = jax.nn.relu((acc_ref[...] + y_ref[...]) * dinv + b_ref[...])
    o_ref[...] = jnp.dot(h, w_ref[...]) * dinv


def _pool_body(acc_ref, y_ref, dinv_ref, b_ref, batch_ref, wfc_ref, bfc_ref,
               o_ref, sacc, scnt):
    i = pl.program_id(0)

    @pl.when(i == 0)
    def _():
        sacc[...] = jnp.zeros_like(sacc)
        scnt[...] = jnp.zeros_like(scnt)

    dinv = dinv_ref[...]
    h = jax.nn.relu((acc_ref[...] + y_ref[...]) * dinv + b_ref[...])
    bid = batch_ref[...].reshape(1, RBLK)
    onehot = (lax.broadcasted_iota(jnp.int32, (B, RBLK), 0) == bid
              ).astype(jnp.float32)
    sacc[...] += jnp.dot(onehot, h)
    scnt[...] += jnp.sum(onehot, axis=1, keepdims=True)

    @pl.when(i == pl.num_programs(0) - 1)
    def _():
        pooled = sacc[...] / jnp.maximum(scnt[...], 1.0)
        o_ref[...] = jax.nn.relu(jnp.dot(pooled, wfc_ref[...]) + bfc_ref[...])


def _y0(xp, deg, W0p):
    return pl.pallas_call(
        _y0_body,
        grid=(GRID,),
        in_specs=[
            pl.BlockSpec((RBLK, 8), lambda i: (i, 0)),
            pl.BlockSpec((RBLK, 1), lambda i: (i, 0)),
            pl.BlockSpec((8, HID), lambda i: (0, 0)),
        ],
        out_specs=[
            pl.BlockSpec((RBLK, HID), lambda i: (i, 0)),
            pl.BlockSpec((RBLK, 1), lambda i: (i, 0)),
        ],
        out_shape=[
            jax.ShapeDtypeStruct((N_PAD, HID), jnp.float32),
            jax.ShapeDtypeStruct((N_PAD, 1), jnp.float32),
        ],
    )(xp, deg, W0p)


def _mid(acc, y, dinv, b, W):
    return pl.pallas_call(
        _mid_body,
        grid=(GRID,),
        in_specs=[
            pl.BlockSpec((RBLK, HID), lambda i: (i, 0)),
            pl.BlockSpec((RBLK, HID), lambda i: (i, 0)),
            pl.BlockSpec((RBLK, 1), lambda i: (i, 0)),
            pl.BlockSpec((1, HID), lambda i: (0, 0)),
            pl.BlockSpec((HID, HID), lambda i: (0, 0)),
        ],
        out_specs=pl.BlockSpec((RBLK, HID), lambda i: (i, 0)),
        out_shape=jax.ShapeDtypeStruct((N_PAD, HID), jnp.float32),
    )(acc, y, dinv, b, W)


def _pool(acc, y, dinv, b, batchp, Wfc, bfc):
    return pl.pallas_call(
        _pool_body,
        grid=(GRID,),
        in_specs=[
            pl.BlockSpec((RBLK, HID), lambda i: (i, 0)),
            pl.BlockSpec((RBLK, HID), lambda i: (i, 0)),
            pl.BlockSpec((RBLK, 1), lambda i: (i, 0)),
            pl.BlockSpec((1, HID), lambda i: (0, 0)),
            pl.BlockSpec((RBLK, 1), lambda i: (i, 0)),
            pl.BlockSpec((HID, EMB), lambda i: (0, 0)),
            pl.BlockSpec((1, EMB), lambda i: (0, 0)),
        ],
        out_specs=pl.BlockSpec((B, EMB), lambda i: (0, 0)),
        out_shape=jax.ShapeDtypeStruct((B, EMB), jnp.float32),
        scratch_shapes=[
            pltpu.VMEM((B, HID), jnp.float32),
            pltpu.VMEM((B, 1), jnp.float32),
        ],
        compiler_params=pltpu.CompilerParams(
            dimension_semantics=("arbitrary",)),
    )(acc, y, dinv, b, batchp, Wfc, bfc)


def kernel(x, edge_index, batch, W0, b0, W1, b1, W2, b2, Wfc, bfc):
    E = edge_index.shape[1]
    f_in = x.shape[1]
    xp = jnp.zeros((N_PAD, 8), jnp.float32).at[:N, :f_in].set(x)
    W0p = jnp.zeros((8, HID), jnp.float32).at[:f_in].set(W0)
    pad = jnp.full((E_PAD - E,), N, jnp.int32)
    src2d = jnp.concatenate([edge_index[0], pad]).reshape(E_PAD // CHUNK, CHUNK)
    dst2d = jnp.concatenate([edge_index[1], pad]).reshape(E_PAD // CHUNK, CHUNK)
    batchp = jnp.concatenate(
        [batch, jnp.full((N_PAD - N,), B, jnp.int32)]).reshape(N_PAD, 1)
    ones_tab = jnp.ones((NQ, N_PAD, QW), jnp.float32)

    deg4 = _edge_accumulate(ones_tab, src2d, dst2d)
    deg = deg4[0, :, :1]
    y0, dinv = _y0(xp, deg, W0p)
    acc0 = _from_quarters(_edge_accumulate(_to_quarters(y0), src2d, dst2d))
    y1 = _mid(acc0, y0, dinv, b0.reshape(1, HID), W1)
    acc1 = _from_quarters(_edge_accumulate(_to_quarters(y1), src2d, dst2d))
    y2 = _mid(acc1, y1, dinv, b1.reshape(1, HID), W2)
    acc2 = _from_quarters(_edge_accumulate(_to_quarters(y2), src2d, dst2d))
    return _pool(acc2, y2, dinv, b2.reshape(1, HID), batchp,
                 Wfc, bfc.reshape(1, EMB))


# speedup vs baseline: 12.0380x; 1.3335x over previous
import jax
import jax.numpy as jnp
from jax import lax
from jax.experimental import pallas as pl
from jax.experimental.pallas import tpu as pltpu
from jax.experimental.pallas import tpu_sc as plsc

N = 50000
B = 16
HID = 64
EMB = 128
NQ = 4
QW = 16
RBLK = 512
N_PAD = 50176
GRID = N_PAD // RBLK
E_PAD = 802816
CHUNK = 128
BLK = 4
ROWS_PER_TILE = N_PAD // 16
CHUNKS_PER_TILE = (E_PAD // CHUNK) // 16
EITERS = CHUNKS_PER_TILE // BLK
ZROWS = 1024


def _zero_slice(zbuf, acc_sh, row0, rem):
    for k in range(3):
        pltpu.sync_copy(zbuf, acc_sh.at[pl.ds(row0 + k * ZROWS, ZROWS)])
    pltpu.sync_copy(zbuf.at[pl.ds(0, rem)],
                    acc_sh.at[pl.ds(row0 + 3 * ZROWS, rem)])


def _copy_out(acc_sh, acc_out, q, row0, rem):
    for k in range(3):
        pltpu.sync_copy(acc_sh.at[pl.ds(row0 + k * ZROWS, ZROWS)],
                        acc_out.at[q, pl.ds(row0 + k * ZROWS, ZROWS)])
    pltpu.sync_copy(acc_sh.at[pl.ds(row0 + 3 * ZROWS, rem)],
                    acc_out.at[q, pl.ds(row0 + 3 * ZROWS, rem)])


def _edge_acc_body(table, src_idx, dst_idx, acc_out,
                   srcb, dstb, rows, zbuf, acc_sh, gsem, ssem):
    c = lax.axis_index("c")
    s = lax.axis_index("s")
    zv = jnp.zeros((QW,), jnp.float32)

    @pl.loop(0, ZROWS)
    def _(i):
        zbuf[i, pl.ds(0, QW)] = zv

    row0 = s * ROWS_PER_TILE
    rem = ROWS_PER_TILE - 3 * ZROWS
    chunk0 = s * CHUNKS_PER_TILE

    def fire_gathers(table_q, buf, it):
        cb = chunk0 + it * BLK
        pltpu.sync_copy(src_idx.at[pl.ds(cb, BLK)], srcb.at[buf])
        pltpu.sync_copy(dst_idx.at[pl.ds(cb, BLK)], dstb.at[buf])
        for j in range(BLK):
            pltpu.async_copy(table_q.at[srcb.at[buf, j]],
                             rows.at[buf].at[pl.ds(j * CHUNK, CHUNK)],
                             gsem.at[buf])

    def drain_gathers(table_q, buf):
        for j in range(BLK):
            pltpu.make_async_copy(
                table_q.at[srcb.at[buf, j]],
                rows.at[buf].at[pl.ds(j * CHUNK, CHUNK)],
                gsem.at[buf]).wait()

    def fire_scatters(buf):
        for j in range(BLK):
            pltpu.async_copy(rows.at[buf].at[pl.ds(j * CHUNK, CHUNK)],
                             acc_sh.at[dstb.at[buf, j]], ssem.at[buf],
                             add=True)

    def drain_scatters(table_q, buf):
        for j in range(BLK):
            pltpu.make_async_copy(
                table_q.at[srcb.at[buf, j]],
                rows.at[buf].at[pl.ds(j * CHUNK, CHUNK)],
                ssem.at[buf]).wait()

    for phase in range(2):
        q = c * 2 + phase
        _zero_slice(zbuf, acc_sh, row0, rem)
        plsc.subcore_barrier()

        table_q = table.at[q]
        fire_gathers(table_q, 0, 0)

        @pl.loop(0, EITERS, step=2)
        def _(it):
            for par in range(2):
                cur, nxt = par, 1 - par
                itc = it + par

                drain_gathers(table_q, cur)
                fire_scatters(cur)

                @pl.when(itc >= 1)
                def _():
                    drain_scatters(table_q, nxt)

                @pl.when(itc + 1 < EITERS)
                def _():
                    fire_gathers(table_q, nxt, itc + 1)

        drain_scatters(table_q, 1)

        plsc.subcore_barrier()
        _copy_out(acc_sh, acc_out, q, row0, rem)
        plsc.subcore_barrier()


def _edge_accumulate(table, src2d, dst2d):
    f = pl.kernel(
        _edge_acc_body,
        out_type=jax.ShapeDtypeStruct((NQ, N_PAD, QW), jnp.float32),
        mesh=plsc.VectorSubcoreMesh(core_axis_name="c", subcore_axis_name="s"),
        scratch_types=[
            pltpu.VMEM((2, BLK, CHUNK), jnp.int32),
            pltpu.VMEM((2, BLK, CHUNK), jnp.int32),
            pltpu.VMEM((2, BLK * CHUNK, QW), jnp.float32),
            pltpu.VMEM((ZROWS, QW), jnp.float32),
            pltpu.VMEM_SHARED((N_PAD, QW), jnp.float32),
            pltpu.SemaphoreType.DMA((2,)),
            pltpu.SemaphoreType.DMA((2,)),
        ],
        compiler_params=pltpu.CompilerParams(use_tc_tiling_on_sc=False),
    )
    return f(table, src2d, dst2d)


DEG_CPT = (E_PAD // CHUNK) // 32
DEG_ITERS = DEG_CPT // BLK


def _deg_body(dst_idx, deg_out, dstb, ones_rows, zbuf, acc_sh):
    c = lax.axis_index("c")
    s = lax.axis_index("s")
    zv = jnp.zeros((QW,), jnp.float32)
    ov = jnp.ones((QW,), jnp.float32)

    @pl.loop(0, ZROWS)
    def _(i):
        zbuf[i, pl.ds(0, QW)] = zv

    @pl.loop(0, CHUNK)
    def _(i):
        ones_rows[i, pl.ds(0, QW)] = ov

    row0 = s * ROWS_PER_TILE
    rem = ROWS_PER_TILE - 3 * ZROWS
    _zero_slice(zbuf, acc_sh, row0, rem)
    plsc.subcore_barrier()

    chunk0 = (c * 16 + s) * DEG_CPT

    @pl.loop(0, DEG_ITERS)
    def _(it):
        cb = chunk0 + it * BLK
        pltpu.sync_copy(dst_idx.at[pl.ds(cb, BLK)], dstb)
        for j in range(BLK):
            pltpu.sync_copy(ones_rows, acc_sh.at[dstb.at[j]], add=True)

    plsc.subcore_barrier()
    _copy_out(acc_sh, deg_out, c, row0, rem)


def _degrees(dst2d):
    f = pl.kernel(
        _deg_body,
        out_type=jax.ShapeDtypeStruct((2, N_PAD, QW), jnp.float32),
        mesh=plsc.VectorSubcoreMesh(core_axis_name="c", subcore_axis_name="s"),
        scratch_types=[
            pltpu.VMEM((BLK, CHUNK), jnp.int32),
            pltpu.VMEM((CHUNK, QW), jnp.float32),
            pltpu.VMEM((ZROWS, QW), jnp.float32),
            pltpu.VMEM_SHARED((N_PAD, QW), jnp.float32),
        ],
        compiler_params=pltpu.CompilerParams(use_tc_tiling_on_sc=False),
    )
    return f(dst2d)


def _to_quarters(y):
    return y.reshape(N_PAD, NQ, QW).transpose(1, 0, 2)


def _from_quarters(a4):
    return a4.transpose(1, 0, 2).reshape(N_PAD, HID)


def _y0_body(x_ref, deg_ref, w_ref, y_ref, dinv_ref):
    xw = jnp.dot(x_ref[...], w_ref[...])
    dinv = lax.rsqrt(deg_ref[...] + 1.0)
    y_ref[...] = xw * dinv
    dinv_ref[...] = dinv


def _mid_body(acc_ref, y_ref, dinv_ref, b_ref, w_ref, o_ref):
    dinv = dinv_ref[...]
    h = jax.nn.relu((acc_ref[...] + y_ref[...]) * dinv + b_ref[...])
    o_ref[...] = jnp.dot(h, w_ref[...]) * dinv


def _pool_body(acc_ref, y_ref, dinv_ref, b_ref, batch_ref, wfc_ref, bfc_ref,
               o_ref, sacc, scnt):
    i = pl.program_id(0)

    @pl.when(i == 0)
    def _():
        sacc[...] = jnp.zeros_like(sacc)
        scnt[...] = jnp.zeros_like(scnt)

    dinv = dinv_ref[...]
    h = jax.nn.relu((acc_ref[...] + y_ref[...]) * dinv + b_ref[...])
    bid = batch_ref[...].reshape(1, RBLK)
    onehot = (lax.broadcasted_iota(jnp.int32, (B, RBLK), 0) == bid
              ).astype(jnp.float32)
    sacc[...] += jnp.dot(onehot, h)
    scnt[...] += jnp.sum(onehot, axis=1, keepdims=True)

    @pl.when(i == pl.num_programs(0) - 1)
    def _():
        pooled = sacc[...] / jnp.maximum(scnt[...], 1.0)
        o_ref[...] = jax.nn.relu(jnp.dot(pooled, wfc_ref[...]) + bfc_ref[...])


def _y0(xp, deg, W0p):
    return pl.pallas_call(
        _y0_body,
        grid=(GRID,),
        in_specs=[
            pl.BlockSpec((RBLK, 8), lambda i: (i, 0)),
            pl.BlockSpec((RBLK, 1), lambda i: (i, 0)),
            pl.BlockSpec((8, HID), lambda i: (0, 0)),
        ],
        out_specs=[
            pl.BlockSpec((RBLK, HID), lambda i: (i, 0)),
            pl.BlockSpec((RBLK, 1), lambda i: (i, 0)),
        ],
        out_shape=[
            jax.ShapeDtypeStruct((N_PAD, HID), jnp.float32),
            jax.ShapeDtypeStruct((N_PAD, 1), jnp.float32),
        ],
    )(xp, deg, W0p)


def _mid(acc, y, dinv, b, W):
    return pl.pallas_call(
        _mid_body,
        grid=(GRID,),
        in_specs=[
            pl.BlockSpec((RBLK, HID), lambda i: (i, 0)),
            pl.BlockSpec((RBLK, HID), lambda i: (i, 0)),
            pl.BlockSpec((RBLK, 1), lambda i: (i, 0)),
            pl.BlockSpec((1, HID), lambda i: (0, 0)),
            pl.BlockSpec((HID, HID), lambda i: (0, 0)),
        ],
        out_specs=pl.BlockSpec((RBLK, HID), lambda i: (i, 0)),
        out_shape=jax.ShapeDtypeStruct((N_PAD, HID), jnp.float32),
    )(acc, y, dinv, b, W)


def _pool(acc, y, dinv, b, batchp, Wfc, bfc):
    return pl.pallas_call(
        _pool_body,
        grid=(GRID,),
        in_specs=[
            pl.BlockSpec((RBLK, HID), lambda i: (i, 0)),
            pl.BlockSpec((RBLK, HID), lambda i: (i, 0)),
            pl.BlockSpec((RBLK, 1), lambda i: (i, 0)),
            pl.BlockSpec((1, HID), lambda i: (0, 0)),
            pl.BlockSpec((RBLK, 1), lambda i: (i, 0)),
            pl.BlockSpec((HID, EMB), lambda i: (0, 0)),
            pl.BlockSpec((1, EMB), lambda i: (0, 0)),
        ],
        out_specs=pl.BlockSpec((B, EMB), lambda i: (0, 0)),
        out_shape=jax.ShapeDtypeStruct((B, EMB), jnp.float32),
        scratch_shapes=[
            pltpu.VMEM((B, HID), jnp.float32),
            pltpu.VMEM((B, 1), jnp.float32),
        ],
        compiler_params=pltpu.CompilerParams(
            dimension_semantics=("arbitrary",)),
    )(acc, y, dinv, b, batchp, Wfc, bfc)


def kernel(x, edge_index, batch, W0, b0, W1, b1, W2, b2, Wfc, bfc):
    E = edge_index.shape[1]
    f_in = x.shape[1]
    xp = jnp.zeros((N_PAD, 8), jnp.float32).at[:N, :f_in].set(x)
    W0p = jnp.zeros((8, HID), jnp.float32).at[:f_in].set(W0)
    pad = jnp.full((E_PAD - E,), N, jnp.int32)
    src2d = jnp.concatenate([edge_index[0], pad]).reshape(E_PAD // CHUNK, CHUNK)
    dst2d = jnp.concatenate([edge_index[1], pad]).reshape(E_PAD // CHUNK, CHUNK)
    batchp = jnp.concatenate(
        [batch, jnp.full((N_PAD - N,), B, jnp.int32)]).reshape(N_PAD, 1)
    degp = _degrees(dst2d)
    deg = degp[0, :, :1] + degp[1, :, :1]
    y0, dinv = _y0(xp, deg, W0p)
    acc0 = _from_quarters(_edge_accumulate(_to_quarters(y0), src2d, dst2d))
    y1 = _mid(acc0, y0, dinv, b0.reshape(1, HID), W1)
    acc1 = _from_quarters(_edge_accumulate(_to_quarters(y1), src2d, dst2d))
    y2 = _mid(acc1, y1, dinv, b1.reshape(1, HID), W2)
    acc2 = _from_quarters(_edge_accumulate(_to_quarters(y2), src2d, dst2d))
    return _pool(acc2, y2, dinv, b2.reshape(1, HID), batchp,
                 Wfc, bfc.reshape(1, EMB))


# speedup vs baseline: 15.4736x; 1.2854x over previous
import jax
import jax.numpy as jnp
from jax import lax
from jax.experimental import pallas as pl
from jax.experimental.pallas import tpu as pltpu
from jax.experimental.pallas import tpu_sc as plsc

N = 50000
B = 16
HID = 64
EMB = 128
NQ = 4
QW = 16
RBLK = 512
N_PAD = 50176
GRID = N_PAD // RBLK
E_PAD = 802816
CHUNK = 128
BLK = 4
ROWS_PER_TILE = N_PAD // 16
CHUNKS_PER_TILE = (E_PAD // CHUNK) // 16
EITERS = CHUNKS_PER_TILE // BLK
ZROWS = 1024


def _zero_slice(zbuf, acc_sh, row0, rem):
    for k in range(3):
        pltpu.sync_copy(zbuf, acc_sh.at[pl.ds(row0 + k * ZROWS, ZROWS)])
    pltpu.sync_copy(zbuf.at[pl.ds(0, rem)],
                    acc_sh.at[pl.ds(row0 + 3 * ZROWS, rem)])


def _copy_out_stripe(acc_sh, acc_out, col, row0, rem):
    for k in range(3):
        pltpu.sync_copy(acc_sh.at[pl.ds(row0 + k * ZROWS, ZROWS)],
                        acc_out.at[pl.ds(row0 + k * ZROWS, ZROWS),
                                   pl.ds(col, QW)])
    pltpu.sync_copy(acc_sh.at[pl.ds(row0 + 3 * ZROWS, rem)],
                    acc_out.at[pl.ds(row0 + 3 * ZROWS, rem),
                               pl.ds(col, QW)])


def _edge_acc_body(table, src_idx, dst_idx, acc_out,
                   srcb, dstb, rows, zbuf, acc_sh, gsem, ssem):
    c = lax.axis_index("c")
    s = lax.axis_index("s")
    zv = jnp.zeros((QW,), jnp.float32)

    @pl.loop(0, ZROWS)
    def _(i):
        zbuf[i, pl.ds(0, QW)] = zv

    row0 = s * ROWS_PER_TILE
    rem = ROWS_PER_TILE - 3 * ZROWS
    chunk0 = s * CHUNKS_PER_TILE

    def fire_gathers(q, buf, it):
        cb = chunk0 + it * BLK
        pltpu.sync_copy(src_idx.at[q, pl.ds(cb, BLK)], srcb.at[buf])
        pltpu.sync_copy(dst_idx.at[pl.ds(cb, BLK)], dstb.at[buf])
        for j in range(BLK):
            pltpu.async_copy(table.at[srcb.at[buf, j]],
                             rows.at[buf].at[pl.ds(j * CHUNK, CHUNK)],
                             gsem.at[buf])

    def drain_gathers(buf):
        for j in range(BLK):
            pltpu.make_async_copy(
                table.at[srcb.at[buf, j]],
                rows.at[buf].at[pl.ds(j * CHUNK, CHUNK)],
                gsem.at[buf]).wait()

    def fire_scatters(buf):
        for j in range(BLK):
            pltpu.async_copy(rows.at[buf].at[pl.ds(j * CHUNK, CHUNK)],
                             acc_sh.at[dstb.at[buf, j]], ssem.at[buf],
                             add=True)

    def drain_scatters(buf):
        for j in range(BLK):
            pltpu.make_async_copy(
                table.at[srcb.at[buf, j]],
                rows.at[buf].at[pl.ds(j * CHUNK, CHUNK)],
                ssem.at[buf]).wait()

    for phase in range(2):
        q = c * 2 + phase
        _zero_slice(zbuf, acc_sh, row0, rem)
        plsc.subcore_barrier()

        fire_gathers(q, 0, 0)

        @pl.loop(0, EITERS, step=2)
        def _(it):
            for par in range(2):
                cur, nxt = par, 1 - par
                itc = it + par

                drain_gathers(cur)
                fire_scatters(cur)

                @pl.when(itc >= 1)
                def _():
                    drain_scatters(nxt)

                @pl.when(itc + 1 < EITERS)
                def _():
                    fire_gathers(q, nxt, itc + 1)

        drain_scatters(1)

        plsc.subcore_barrier()
        _copy_out_stripe(acc_sh, acc_out, q * QW, row0, rem)
        plsc.subcore_barrier()


def _edge_accumulate(table8, srcq, dst2d):
    f = pl.kernel(
        _edge_acc_body,
        out_type=jax.ShapeDtypeStruct((N_PAD, 8 * QW), jnp.float32),
        mesh=plsc.VectorSubcoreMesh(core_axis_name="c", subcore_axis_name="s"),
        scratch_types=[
            pltpu.VMEM((2, BLK, CHUNK), jnp.int32),
            pltpu.VMEM((2, BLK, CHUNK), jnp.int32),
            pltpu.VMEM((2, BLK * CHUNK, QW), jnp.float32),
            pltpu.VMEM((ZROWS, QW), jnp.float32),
            pltpu.VMEM_SHARED((N_PAD, QW), jnp.float32),
            pltpu.SemaphoreType.DMA((2,)),
            pltpu.SemaphoreType.DMA((2,)),
        ],
        compiler_params=pltpu.CompilerParams(use_tc_tiling_on_sc=False),
    )
    return f(table8, srcq, dst2d)


DEG_CPT = (E_PAD // CHUNK) // 32
DBLK = 4
DEG_ITERS = DEG_CPT // DBLK


def _deg_body(dst_idx, deg_out, dstb, ones_rows, zbuf, acc_sh):
    c = lax.axis_index("c")
    s = lax.axis_index("s")
    zv = jnp.zeros((QW,), jnp.float32)
    ov = jnp.ones((QW,), jnp.float32)

    @pl.loop(0, ZROWS)
    def _(i):
        zbuf[i, pl.ds(0, QW)] = zv

    @pl.loop(0, CHUNK)
    def _(i):
        ones_rows[i, pl.ds(0, QW)] = ov

    row0 = s * ROWS_PER_TILE
    rem = ROWS_PER_TILE - 3 * ZROWS
    _zero_slice(zbuf, acc_sh, row0, rem)
    plsc.subcore_barrier()

    chunk0 = (c * 16 + s) * DEG_CPT

    @pl.loop(0, DEG_ITERS)
    def _(it):
        cb = chunk0 + it * DBLK
        pltpu.sync_copy(dst_idx.at[pl.ds(cb, DBLK)], dstb)
        for j in range(DBLK):
            pltpu.sync_copy(ones_rows, acc_sh.at[dstb.at[j]], add=True)

    plsc.subcore_barrier()
    _copy_out_stripe(acc_sh, deg_out, c * QW, row0, rem)


def _degrees(dst2d):
    f = pl.kernel(
        _deg_body,
        out_type=jax.ShapeDtypeStruct((N_PAD, 8 * QW), jnp.float32),
        mesh=plsc.VectorSubcoreMesh(core_axis_name="c", subcore_axis_name="s"),
        scratch_types=[
            pltpu.VMEM((DBLK, CHUNK), jnp.int32),
            pltpu.VMEM((CHUNK, QW), jnp.float32),
            pltpu.VMEM((ZROWS, QW), jnp.float32),
            pltpu.VMEM_SHARED((N_PAD, QW), jnp.float32),
        ],
        compiler_params=pltpu.CompilerParams(use_tc_tiling_on_sc=False),
    )
    return f(dst2d)


def _y0_body(x_ref, deg_ref, w_ref, y_ref):
    xw = jnp.dot(x_ref[...], w_ref[...])
    deg = deg_ref[:, 0:1] + deg_ref[:, QW:QW + 1]
    dinv = lax.rsqrt(deg + 1.0)
    y_ref[...] = jnp.concatenate(
        [xw * dinv, jnp.broadcast_to(dinv, (RBLK, HID))], axis=1)


def _mid_body(acc_ref, y_ref, b_ref, w_ref, o_ref):
    dinv = y_ref[:, HID:HID + 1]
    h = jax.nn.relu((acc_ref[:, :HID] + y_ref[:, :HID]) * dinv + b_ref[...])
    yn = jnp.dot(h, w_ref[...]) * dinv
    o_ref[...] = jnp.concatenate(
        [yn, jnp.broadcast_to(dinv, (RBLK, HID))], axis=1)


def _pool_body(acc_ref, y_ref, b_ref, batch_ref, wfc_ref, bfc_ref,
               o_ref, sacc, scnt):
    i = pl.program_id(0)

    @pl.when(i == 0)
    def _():
        sacc[...] = jnp.zeros_like(sacc)
        scnt[...] = jnp.zeros_like(scnt)

    dinv = y_ref[:, HID:HID + 1]
    h = jax.nn.relu((acc_ref[:, :HID] + y_ref[:, :HID]) * dinv + b_ref[...])
    bid = batch_ref[...].reshape(1, RBLK)
    onehot = (lax.broadcasted_iota(jnp.int32, (B, RBLK), 0) == bid
              ).astype(jnp.float32)
    sacc[...] += jnp.dot(onehot, h)
    scnt[...] += jnp.sum(onehot, axis=1, keepdims=True)

    @pl.when(i == pl.num_programs(0) - 1)
    def _():
        pooled = sacc[...] / jnp.maximum(scnt[...], 1.0)
        o_ref[...] = jax.nn.relu(jnp.dot(pooled, wfc_ref[...]) + bfc_ref[...])


def _y0(xp, deg128, W0p):
    return pl.pallas_call(
        _y0_body,
        grid=(GRID,),
        in_specs=[
            pl.BlockSpec((RBLK, 8), lambda i: (i, 0)),
            pl.BlockSpec((RBLK, 128), lambda i: (i, 0)),
            pl.BlockSpec((8, HID), lambda i: (0, 0)),
        ],
        out_specs=pl.BlockSpec((RBLK, 128), lambda i: (i, 0)),
        out_shape=jax.ShapeDtypeStruct((N_PAD, 128), jnp.float32),
    )(xp, deg128, W0p)


def _mid(acc, y, b, W):
    return pl.pallas_call(
        _mid_body,
        grid=(GRID,),
        in_specs=[
            pl.BlockSpec((RBLK, 128), lambda i: (i, 0)),
            pl.BlockSpec((RBLK, 128), lambda i: (i, 0)),
            pl.BlockSpec((1, HID), lambda i: (0, 0)),
            pl.BlockSpec((HID, HID), lambda i: (0, 0)),
        ],
        out_specs=pl.BlockSpec((RBLK, 128), lambda i: (i, 0)),
        out_shape=jax.ShapeDtypeStruct((N_PAD, 128), jnp.float32),
    )(acc, y, b, W)


def _pool(acc, y, b, batchp, Wfc, bfc):
    return pl.pallas_call(
        _pool_body,
        grid=(GRID,),
        in_specs=[
            pl.BlockSpec((RBLK, 128), lambda i: (i, 0)),
            pl.BlockSpec((RBLK, 128), lambda i: (i, 0)),
            pl.BlockSpec((1, HID), lambda i: (0, 0)),
            pl.BlockSpec((RBLK, 1), lambda i: (i, 0)),
            pl.BlockSpec((HID, EMB), lambda i: (0, 0)),
            pl.BlockSpec((1, EMB), lambda i: (0, 0)),
        ],
        out_specs=pl.BlockSpec((B, EMB), lambda i: (0, 0)),
        out_shape=jax.ShapeDtypeStruct((B, EMB), jnp.float32),
        scratch_shapes=[
            pltpu.VMEM((B, HID), jnp.float32),
            pltpu.VMEM((B, 1), jnp.float32),
        ],
        compiler_params=pltpu.CompilerParams(
            dimension_semantics=("arbitrary",)),
    )(acc, y, b, batchp, Wfc, bfc)


def kernel(x, edge_index, batch, W0, b0, W1, b1, W2, b2, Wfc, bfc):
    E = edge_index.shape[1]
    f_in = x.shape[1]
    xp = jnp.zeros((N_PAD, 8), jnp.float32).at[:N, :f_in].set(x)
    W0p = jnp.zeros((8, HID), jnp.float32).at[:f_in].set(W0)
    pad = jnp.full((E_PAD - E,), N, jnp.int32)
    srcp = jnp.concatenate([edge_index[0], pad])
    srcq = (srcp[None, :] * 8 + jnp.arange(NQ, dtype=jnp.int32)[:, None]
            ).reshape(NQ, E_PAD // CHUNK, CHUNK)
    dst2d = jnp.concatenate([edge_index[1], pad]).reshape(E_PAD // CHUNK, CHUNK)
    batchp = jnp.concatenate(
        [batch, jnp.full((N_PAD - N,), B, jnp.int32)]).reshape(N_PAD, 1)

    deg128 = _degrees(dst2d)
    y0 = _y0(xp, deg128, W0p)
    acc0 = _edge_accumulate(y0.reshape(8 * N_PAD, QW), srcq, dst2d)
    y1 = _mid(acc0, y0, b0.reshape(1, HID), W1)
    acc1 = _edge_accumulate(y1.reshape(8 * N_PAD, QW), srcq, dst2d)
    y2 = _mid(acc1, y1, b1.reshape(1, HID), W2)
    acc2 = _edge_accumulate(y2.reshape(8 * N_PAD, QW), srcq, dst2d)
    return _pool(acc2, y2, b2.reshape(1, HID), batchp,
                 Wfc, bfc.reshape(1, EMB))


# speedup vs baseline: 18.6731x; 1.2068x over previous
import jax
import jax.numpy as jnp
from jax import lax
from jax.experimental import pallas as pl
from jax.experimental.pallas import tpu as pltpu
from jax.experimental.pallas import tpu_sc as plsc

N = 50000
B = 16
HID = 64
EMB = 128
NQ = 4
QW = 16
RBLK = 1024
N_PAD = 50176
GRID = N_PAD // RBLK
E_PAD = 802816
CHUNK = 128
BLK = 4
ROWS_PER_TILE = N_PAD // 16
CHUNKS_PER_TILE = (E_PAD // CHUNK) // 16
EITERS = CHUNKS_PER_TILE // BLK
ZROWS = 1024


def _zero_slice(zbuf, acc_sh, row0, rem):
    for k in range(3):
        pltpu.sync_copy(zbuf, acc_sh.at[pl.ds(row0 + k * ZROWS, ZROWS)])
    pltpu.sync_copy(zbuf.at[pl.ds(0, rem)],
                    acc_sh.at[pl.ds(row0 + 3 * ZROWS, rem)])


def _copy_out_stripe(acc_sh, acc_out, col, row0, rem):
    for k in range(3):
        pltpu.sync_copy(acc_sh.at[pl.ds(row0 + k * ZROWS, ZROWS)],
                        acc_out.at[pl.ds(row0 + k * ZROWS, ZROWS),
                                   pl.ds(col, QW)])
    pltpu.sync_copy(acc_sh.at[pl.ds(row0 + 3 * ZROWS, rem)],
                    acc_out.at[pl.ds(row0 + 3 * ZROWS, rem),
                               pl.ds(col, QW)])


def _edge_acc_body(table, sd_idx, acc_out,
                   sdb, rows, zbuf, acc_sh, gsem, ssem):
    c = lax.axis_index("c")
    s = lax.axis_index("s")
    zv = jnp.zeros((QW,), jnp.float32)

    @pl.loop(0, ZROWS)
    def _(i):
        zbuf[i, pl.ds(0, QW)] = zv

    row0 = s * ROWS_PER_TILE
    rem = ROWS_PER_TILE - 3 * ZROWS
    chunk0 = s * CHUNKS_PER_TILE

    def fire_gathers(q, buf, it):
        cb = chunk0 + it * BLK
        pltpu.sync_copy(sd_idx.at[q, pl.ds(cb, BLK)], sdb.at[buf])
        for j in range(BLK):
            pltpu.async_copy(table.at[sdb.at[buf, j, 0]],
                             rows.at[buf].at[pl.ds(j * CHUNK, CHUNK)],
                             gsem.at[buf])

    def drain_gathers(buf):
        for j in range(BLK):
            pltpu.make_async_copy(
                table.at[sdb.at[buf, j, 0]],
                rows.at[buf].at[pl.ds(j * CHUNK, CHUNK)],
                gsem.at[buf]).wait()

    def fire_scatters(buf):
        for j in range(BLK):
            pltpu.async_copy(rows.at[buf].at[pl.ds(j * CHUNK, CHUNK)],
                             acc_sh.at[sdb.at[buf, j, 1]], ssem.at[buf],
                             add=True)

    def drain_scatters(buf):
        for j in range(BLK):
            pltpu.make_async_copy(
                table.at[sdb.at[buf, j, 0]],
                rows.at[buf].at[pl.ds(j * CHUNK, CHUNK)],
                ssem.at[buf]).wait()

    for phase in range(2):
        q = c * 2 + phase
        _zero_slice(zbuf, acc_sh, row0, rem)
        plsc.subcore_barrier()

        fire_gathers(q, 0, 0)

        @pl.loop(0, EITERS, step=2)
        def _(it):
            for par in range(2):
                cur, nxt = par, 1 - par
                itc = it + par

                drain_gathers(cur)
                fire_scatters(cur)

                @pl.when(itc >= 1)
                def _():
                    drain_scatters(nxt)

                @pl.when(itc + 1 < EITERS)
                def _():
                    fire_gathers(q, nxt, itc + 1)

        drain_scatters(1)

        plsc.subcore_barrier()
        _copy_out_stripe(acc_sh, acc_out, q * QW, row0, rem)
        plsc.subcore_barrier()


def _edge_accumulate(table8, srcdst, dst2d):
    f = pl.kernel(
        _edge_acc_body,
        out_type=jax.ShapeDtypeStruct((N_PAD, 8 * QW), jnp.float32),
        mesh=plsc.VectorSubcoreMesh(core_axis_name="c", subcore_axis_name="s"),
        scratch_types=[
            pltpu.VMEM((2, BLK, 2, CHUNK), jnp.int32),
            pltpu.VMEM((2, BLK * CHUNK, QW), jnp.float32),
            pltpu.VMEM((ZROWS, QW), jnp.float32),
            pltpu.VMEM_SHARED((N_PAD, QW), jnp.float32),
            pltpu.SemaphoreType.DMA((2,)),
            pltpu.SemaphoreType.DMA((2,)),
        ],
        compiler_params=pltpu.CompilerParams(use_tc_tiling_on_sc=False),
    )
    return f(table8, srcdst)


DEG_CPT = (E_PAD // CHUNK) // 32
DBLK = 4
DEG_ITERS = DEG_CPT // DBLK


def _deg_body(dst_idx, deg_out, dstb, ones_rows, zbuf, acc_sh):
    c = lax.axis_index("c")
    s = lax.axis_index("s")
    zv = jnp.zeros((QW,), jnp.float32)
    ov = jnp.ones((QW,), jnp.float32)

    @pl.loop(0, ZROWS)
    def _(i):
        zbuf[i, pl.ds(0, QW)] = zv

    @pl.loop(0, CHUNK)
    def _(i):
        ones_rows[i, pl.ds(0, QW)] = ov

    row0 = s * ROWS_PER_TILE
    rem = ROWS_PER_TILE - 3 * ZROWS
    _zero_slice(zbuf, acc_sh, row0, rem)
    plsc.subcore_barrier()

    chunk0 = (c * 16 + s) * DEG_CPT

    @pl.loop(0, DEG_ITERS)
    def _(it):
        cb = chunk0 + it * DBLK
        pltpu.sync_copy(dst_idx.at[pl.ds(cb, DBLK)], dstb)
        for j in range(DBLK):
            pltpu.sync_copy(ones_rows, acc_sh.at[dstb.at[j]], add=True)

    plsc.subcore_barrier()
    _copy_out_stripe(acc_sh, deg_out, c * QW, row0, rem)


def _degrees(dst2d):
    f = pl.kernel(
        _deg_body,
        out_type=jax.ShapeDtypeStruct((N_PAD, 8 * QW), jnp.float32),
        mesh=plsc.VectorSubcoreMesh(core_axis_name="c", subcore_axis_name="s"),
        scratch_types=[
            pltpu.VMEM((DBLK, CHUNK), jnp.int32),
            pltpu.VMEM((CHUNK, QW), jnp.float32),
            pltpu.VMEM((ZROWS, QW), jnp.float32),
            pltpu.VMEM_SHARED((N_PAD, QW), jnp.float32),
        ],
        compiler_params=pltpu.CompilerParams(use_tc_tiling_on_sc=False),
    )
    return f(dst2d)


def _y0_body(x_ref, deg_ref, w_ref, y_ref):
    xw = jnp.dot(x_ref[...], w_ref[...])
    deg = deg_ref[:, 0:1] + deg_ref[:, QW:QW + 1]
    dinv = lax.rsqrt(deg + 1.0)
    y_ref[...] = jnp.concatenate(
        [xw * dinv, jnp.broadcast_to(dinv, (RBLK, HID))], axis=1)


def _mid_body(acc_ref, y_ref, b_ref, w_ref, o_ref):
    dinv = y_ref[:, HID:HID + 1]
    h = jax.nn.relu((acc_ref[:, :HID] + y_ref[:, :HID]) * dinv + b_ref[...])
    yn = jnp.dot(h, w_ref[...]) * dinv
    o_ref[...] = jnp.concatenate(
        [yn, jnp.broadcast_to(dinv, (RBLK, HID))], axis=1)


def _pool_body(acc_ref, y_ref, b_ref, batch_ref, wfc_ref, bfc_ref,
               o_ref, sacc, scnt):
    i = pl.program_id(0)

    @pl.when(i == 0)
    def _():
        sacc[...] = jnp.zeros_like(sacc)
        scnt[...] = jnp.zeros_like(scnt)

    dinv = y_ref[:, HID:HID + 1]
    h = jax.nn.relu((acc_ref[:, :HID] + y_ref[:, :HID]) * dinv + b_ref[...])
    bid = batch_ref[...].reshape(1, RBLK)
    onehot = (lax.broadcasted_iota(jnp.int32, (B, RBLK), 0) == bid
              ).astype(jnp.float32)
    sacc[...] += jnp.dot(onehot, h)
    scnt[...] += jnp.sum(onehot, axis=1, keepdims=True)

    @pl.when(i == pl.num_programs(0) - 1)
    def _():
        pooled = sacc[...] / jnp.maximum(scnt[...], 1.0)
        o_ref[...] = jax.nn.relu(jnp.dot(pooled, wfc_ref[...]) + bfc_ref[...])


def _y0(xp, deg128, W0p):
    return pl.pallas_call(
        _y0_body,
        grid=(GRID,),
        in_specs=[
            pl.BlockSpec((RBLK, 8), lambda i: (i, 0)),
            pl.BlockSpec((RBLK, 128), lambda i: (i, 0)),
            pl.BlockSpec((8, HID), lambda i: (0, 0)),
        ],
        out_specs=pl.BlockSpec((RBLK, 128), lambda i: (i, 0)),
        out_shape=jax.ShapeDtypeStruct((N_PAD, 128), jnp.float32),
    )(xp, deg128, W0p)


def _mid(acc, y, b, W):
    return pl.pallas_call(
        _mid_body,
        grid=(GRID,),
        in_specs=[
            pl.BlockSpec((RBLK, 128), lambda i: (i, 0)),
            pl.BlockSpec((RBLK, 128), lambda i: (i, 0)),
            pl.BlockSpec((1, HID), lambda i: (0, 0)),
            pl.BlockSpec((HID, HID), lambda i: (0, 0)),
        ],
        out_specs=pl.BlockSpec((RBLK, 128), lambda i: (i, 0)),
        out_shape=jax.ShapeDtypeStruct((N_PAD, 128), jnp.float32),
    )(acc, y, b, W)


def _pool(acc, y, b, batchp, Wfc, bfc):
    return pl.pallas_call(
        _pool_body,
        grid=(GRID,),
        in_specs=[
            pl.BlockSpec((RBLK, 128), lambda i: (i, 0)),
            pl.BlockSpec((RBLK, 128), lambda i: (i, 0)),
            pl.BlockSpec((1, HID), lambda i: (0, 0)),
            pl.BlockSpec((RBLK, 1), lambda i: (i, 0)),
            pl.BlockSpec((HID, EMB), lambda i: (0, 0)),
            pl.BlockSpec((1, EMB), lambda i: (0, 0)),
        ],
        out_specs=pl.BlockSpec((B, EMB), lambda i: (0, 0)),
        out_shape=jax.ShapeDtypeStruct((B, EMB), jnp.float32),
        scratch_shapes=[
            pltpu.VMEM((B, HID), jnp.float32),
            pltpu.VMEM((B, 1), jnp.float32),
        ],
        compiler_params=pltpu.CompilerParams(
            dimension_semantics=("arbitrary",)),
    )(acc, y, b, batchp, Wfc, bfc)


def kernel(x, edge_index, batch, W0, b0, W1, b1, W2, b2, Wfc, bfc):
    E = edge_index.shape[1]
    f_in = x.shape[1]
    xp = jnp.zeros((N_PAD, 8), jnp.float32).at[:N, :f_in].set(x)
    W0p = jnp.zeros((8, HID), jnp.float32).at[:f_in].set(W0)
    pad = jnp.full((E_PAD - E,), N, jnp.int32)
    srcp = jnp.concatenate([edge_index[0], pad])
    srcq = (srcp[None, :] * 8 + jnp.arange(NQ, dtype=jnp.int32)[:, None]
            ).reshape(NQ, E_PAD // CHUNK, CHUNK)
    dst2d = jnp.concatenate([edge_index[1], pad]).reshape(E_PAD // CHUNK, CHUNK)
    srcdst = jnp.stack(
        [srcq, jnp.broadcast_to(dst2d, (NQ,) + dst2d.shape)], axis=2)
    batchp = jnp.concatenate(
        [batch, jnp.full((N_PAD - N,), B, jnp.int32)]).reshape(N_PAD, 1)

    deg128 = _degrees(dst2d)
    y0 = _y0(xp, deg128, W0p)
    acc0 = _edge_accumulate(y0.reshape(8 * N_PAD, QW), srcdst, dst2d)
    y1 = _mid(acc0, y0, b0.reshape(1, HID), W1)
    acc1 = _edge_accumulate(y1.reshape(8 * N_PAD, QW), srcdst, dst2d)
    y2 = _mid(acc1, y1, b1.reshape(1, HID), W2)
    acc2 = _edge_accumulate(y2.reshape(8 * N_PAD, QW), srcdst, dst2d)
    return _pool(acc2, y2, b2.reshape(1, HID), batchp,
                 Wfc, bfc.reshape(1, EMB))
